# baseline (speedup 1.0000x reference)
import jax
import jax.numpy as jnp
from jax import lax
from jax.experimental import pallas as pl
from jax.experimental.pallas import tpu as pltpu
from jax.experimental.pallas import tpu_sc as plsc

NUM_CLASSES = 100000
EMBED_DIM = 64
BATCH = 16384

_NC = 2
_NS = 16
_NW = _NC * _NS
_ROWS_PER_W = BATCH // _NW
_GCHUNK = 128
_NG = _ROWS_PER_W // _GCHUNK
_EROWS = _ROWS_PER_W // 2


def _center_loss_body(emb_hbm, tgt_hbm, tbl_hbm, out_hbm,
                      idx_v, widx_v, tflat_v, emb_v, rows_v, out_v,
                      gsem, esem):
    wid = lax.axis_index("s") * _NC + lax.axis_index("c")

    pltpu.sync_copy(tgt_hbm.at[wid], idx_v)

    for j in range(_NG):
        for k in range(_GCHUNK // 16):
            t16 = idx_v[j, pl.ds(k * 16, 16)]
            widx_v[j, pl.ds(k * 16, 16)] = t16 >> 1
            tflat_v[pl.ds(j * _GCHUNK + k * 16, 16)] = t16

    gathers = []
    for j in range(_NG):
        gathers.append(pltpu.async_copy(
            tbl_hbm.at[widx_v.at[j]],
            rows_v.at[pl.ds(j * _GCHUNK, _GCHUNK)],
            gsem))
    emb_cp = pltpu.async_copy(emb_hbm.at[pl.ds(wid * _EROWS, _EROWS)],
                              emb_v, esem)
    emb_cp.wait()
    for g in gathers:
        g.wait()

    zero = jnp.zeros((16,), jnp.float32)

    def body(rr, accs):
        new = list(accs)
        for half in range(2):
            r = rr * 2 + half
            t = tflat_v[pl.ds(r, 16)][0]
            cbase = (t & 1) << 6
            for j in range(4):
                e = emb_v[rr, pl.ds(half * 64 + j * 16, 16)]
                c = rows_v[r, pl.ds(cbase + j * 16, 16)]
                d = e - c
                new[j] = new[j] + d * d
        return tuple(new)

    accs = lax.fori_loop(0, _EROWS, body, (zero, zero, zero, zero))
    total = (accs[0] + accs[1]) + (accs[2] + accs[3])
    out_v[...] = total * jnp.float32(1.0 / BATCH)
    pltpu.sync_copy(out_v, out_hbm.at[wid])


@jax.jit
def _center_loss(embeddings, targets, centers):
    tgt = targets.astype(jnp.int32).reshape(_NW, _NG, _GCHUNK)
    emb2 = embeddings.reshape(BATCH // 2, 2 * EMBED_DIM)
    ctr2 = centers.reshape(NUM_CLASSES // 2, 2 * EMBED_DIM)
    mesh = plsc.VectorSubcoreMesh(core_axis_name="c", subcore_axis_name="s")
    partials = pl.kernel(
        _center_loss_body,
        mesh=mesh,
        out_type=jax.ShapeDtypeStruct((_NW, 16), jnp.float32),
        scratch_types=[
            pltpu.VMEM((_NG, _GCHUNK), jnp.int32),
            pltpu.VMEM((_NG, _GCHUNK), jnp.int32),
            pltpu.VMEM((_ROWS_PER_W + 16, ), jnp.int32),
            pltpu.VMEM((_EROWS, 2 * EMBED_DIM), jnp.float32),
            pltpu.VMEM((_ROWS_PER_W, 2 * EMBED_DIM), jnp.float32),
            pltpu.VMEM((16,), jnp.float32),
            pltpu.SemaphoreType.DMA,
            pltpu.SemaphoreType.DMA,
        ],
    )(emb2, tgt, ctr2)
    return jnp.sum(partials)


def kernel(embeddings, targets, centers):
    return _center_loss(embeddings, targets, centers)
